# baseline (speedup 1.0000x reference)
import jax
import jax.numpy as jnp
from jax import lax
from jax.experimental import pallas as pl
from jax.experimental.pallas import tpu as pltpu
from jax.experimental.pallas import tpu_sc as plsc

NC, NS, L = 2, 16, 16
NW = NC * NS

N_PTS = 1048576
K = 1024
PER_W = N_PTS // NW
NCHUNK = PER_W // K

R = 130
LO = (R - 1) // 2
NCELL = R - 1 - LO
NZ = R - LO
ROWS_PER_Z = NCELL * NCELL
SCALE = float(R - 1)


def _build_table(grid):
    sub = lax.slice(grid, (0, LO, LO, LO), grid.shape)
    subt = jnp.transpose(sub, (1, 2, 3, 0))
    t = jnp.concatenate([
        subt[:, :-1, :-1, :], subt[:, :-1, 1:, :],
        subt[:, 1:, :-1, :], subt[:, 1:, 1:, :]], axis=-1)
    return t.reshape(NZ * ROWS_PER_Z, 16)


def _body(x_hbm, tab_hbm, out_hbm,
          coords_v, idxa_v, idxb_v, rowsa_v, rowsb_v, outb_v, sem):
    wid = lax.axis_index("s") * NC + lax.axis_index("c")
    base = wid * PER_W
    iota = lax.iota(jnp.int32, L)

    def lane_coord(rows, d):
        cv = plsc.load_gather(coords_v, [rows, jnp.full((L,), d, jnp.int32)])
        return (cv + 1.0) * 0.5 * SCALE

    @pl.loop(0, NCHUNK)
    def _chunk(c):
        cbase = base + c * K
        pltpu.sync_copy(x_hbm.at[pl.ds(cbase, K), :], coords_v)

        @pl.loop(0, K // L)
        def _idx(g):
            rows = g * L + iota

            def cell(d):
                iv = lane_coord(rows, d)
                return jnp.minimum(iv.astype(jnp.int32), R - 2) - LO

            xr = cell(0)
            yr = cell(1)
            zr = cell(2)
            r0 = (zr * NCELL + yr) * NCELL + xr
            idxa_v[pl.ds(g * L, L)] = r0
            idxb_v[pl.ds(g * L, L)] = r0 + ROWS_PER_Z

        cpa = pltpu.async_copy(tab_hbm.at[idxa_v], rowsa_v, sem)
        cpb = pltpu.async_copy(tab_hbm.at[idxb_v], rowsb_v, sem)
        cpa.wait()
        cpb.wait()

        @pl.loop(0, K // L)
        def _mac(g):
            rows = g * L + iota

            def frac(d):
                iv = lane_coord(rows, d)
                fi = jnp.minimum(iv.astype(jnp.int32), R - 2)
                return iv - fi.astype(jnp.float32)

            fx = frac(0)
            fy = frac(1)
            fz = frac(2)
            ux = 1.0 - fx
            uy = 1.0 - fy
            uz = 1.0 - fz
            m = (uy * ux, uy * fx, fy * ux, fy * fx)
            acc = [None] * 4
            for rv, wz_ in ((rowsa_v, uz), (rowsb_v, fz)):
                w = [wz_ * mk for mk in m]
                for k4 in range(4):
                    for ch in range(4):
                        col = jnp.full((L,), k4 * 4 + ch, jnp.int32)
                        v = plsc.load_gather(rv, [rows, col])
                        t = w[k4] * v
                        acc[ch] = t if acc[ch] is None else acc[ch] + t
            for ch in range(4):
                plsc.store_scatter(
                    outb_v, [rows, jnp.full((L,), ch, jnp.int32)], acc[ch])

        pltpu.sync_copy(outb_v, out_hbm.at[pl.ds(cbase, K), :])


def kernel(x, grid):
    tab = _build_table(grid)
    mesh = plsc.VectorSubcoreMesh(core_axis_name="c", subcore_axis_name="s")
    run = pl.kernel(
        _body,
        out_type=jax.ShapeDtypeStruct((N_PTS, 4), jnp.float32),
        mesh=mesh,
        scratch_types=[
            pltpu.VMEM((K, 3), jnp.float32),
            pltpu.VMEM((K,), jnp.int32),
            pltpu.VMEM((K,), jnp.int32),
            pltpu.VMEM((K, 16), jnp.float32),
            pltpu.VMEM((K, 16), jnp.float32),
            pltpu.VMEM((K, 4), jnp.float32),
            pltpu.SemaphoreType.DMA,
        ],
        compiler_params=pltpu.CompilerParams(
            needs_layout_passes=False, use_tc_tiling_on_sc=False),
    )
    return run(x, tab)
